# baseline (speedup 1.0000x reference)
import functools

import jax
import jax.numpy as jnp
from jax import lax
from jax.experimental import pallas as pl
from jax.experimental.pallas import tpu as pltpu
from jax.experimental.pallas import tpu_sc as plsc

_NC = 2
_NS = 16
_NW = _NC * _NS
_RS = 32
_S1 = 6144


def _add_body(x_ref, p_ref, o_ref):
    o_ref[...] = x_ref[...] + p_ref[...]


def _sc_body(B, S_off, S_len, D, x_hbm, tab_hbm, out_hbm,
             tbuf, xb0, xb1, sin0, sin1, sout0, sout1):
    wid = lax.axis_index("s") * _NC + lax.axis_index("c")
    rows_per_w = S_len // _NW
    base = wid * rows_per_w
    groups = D // 16
    xbufs = (xb0, xb1)
    sins = (sin0, sin1)
    souts = (sout0, sout1)

    def chunk(i, carry):
        s0 = base + i * _RS
        rows_in = pl.ds(S_off + s0, _RS)
        rows_out = pl.ds(s0, _RS)
        pltpu.sync_copy(tab_hbm.at[rows_in], tbuf)

        loads = [None, None]
        stores = [None, None]
        loads[0] = pltpu.async_copy(x_hbm.at[0, rows_in], xbufs[0], sins[0])
        for b in range(B):
            cur = b % 2
            nxt = (b + 1) % 2
            if b + 1 < B:
                if stores[nxt] is not None:
                    stores[nxt].wait()
                    stores[nxt] = None
                loads[nxt] = pltpu.async_copy(
                    x_hbm.at[b + 1, rows_in], xbufs[nxt], sins[nxt])
            loads[cur].wait()

            xbuf = xbufs[cur]

            @plsc.parallel_loop(0, _RS, 1, unroll=2)
            def row(r):
                for j in range(groups):
                    t = tbuf[r, pl.ds(j * 16, 16)]
                    plsc.addupdate(xbuf.at[r, pl.ds(j * 16, 16)], t)
            stores[cur] = pltpu.async_copy(
                xbufs[cur], out_hbm.at[b, rows_out], souts[cur])
        for d in stores:
            if d is not None:
                d.wait()
        return carry

    lax.fori_loop(0, rows_per_w // _RS, chunk, 0)


def kernel(input_embeddings, pos_table):
    B, S, D = input_embeddings.shape
    TS = 2048
    out_tc = pl.pallas_call(
        _add_body,
        grid=(_S1 // TS, B),
        in_specs=[
            pl.BlockSpec((1, TS, D), lambda s, b: (b, s, 0)),
            pl.BlockSpec((TS, D), lambda s, b: (s, 0)),
        ],
        out_specs=pl.BlockSpec((1, TS, D), lambda s, b: (b, s, 0)),
        out_shape=jax.ShapeDtypeStruct((B, _S1, D), input_embeddings.dtype),
    )(input_embeddings, pos_table)

    S2 = S - _S1
    mesh = plsc.VectorSubcoreMesh(core_axis_name="c", subcore_axis_name="s")
    sc_add = pl.kernel(
        functools.partial(_sc_body, B, _S1, S2, D),
        out_type=jax.ShapeDtypeStruct((B, S2, D), input_embeddings.dtype),
        mesh=mesh,
        scratch_types=[
            pltpu.VMEM((_RS, D), jnp.float32),
            pltpu.VMEM((_RS, D), jnp.float32),
            pltpu.VMEM((_RS, D), jnp.float32),
            pltpu.SemaphoreType.DMA,
            pltpu.SemaphoreType.DMA,
            pltpu.SemaphoreType.DMA,
            pltpu.SemaphoreType.DMA,
        ],
    )
    out_sc = sc_add(input_embeddings, pos_table)
    return jnp.concatenate([out_tc, out_sc], axis=1)


# speedup vs baseline: 2.0989x; 2.0989x over previous
import jax
import jax.numpy as jnp
from jax.experimental import pallas as pl


def _add_body(x_ref, p_ref, o_ref):
    o_ref[...] = x_ref[...] + p_ref[...]


def kernel(input_embeddings, pos_table):
    B, S, D = input_embeddings.shape
    TS = 2048
    grid = (S // TS, B)
    return pl.pallas_call(
        _add_body,
        grid=grid,
        in_specs=[
            pl.BlockSpec((1, TS, D), lambda s, b: (b, s, 0)),
            pl.BlockSpec((TS, D), lambda s, b: (s, 0)),
        ],
        out_specs=pl.BlockSpec((1, TS, D), lambda s, b: (b, s, 0)),
        out_shape=jax.ShapeDtypeStruct((B, S, D), input_embeddings.dtype),
    )(input_embeddings, pos_table)
